# Initial kernel scaffold; baseline (speedup 1.0000x reference)
import jax
import jax.numpy as jnp
from jax.experimental import pallas as pl


def kernel(indices, table):
    raise NotImplementedError("write your pallas kernel here")


# speedup vs baseline: 4.0850x; 4.0850x over previous
import functools

import jax
import jax.numpy as jnp
from jax import lax
from jax.experimental import pallas as pl
from jax.experimental.pallas import tpu as pltpu
from jax.experimental.pallas import tpu_sc as plsc

_NW = 32
_CHUNK = 128


@functools.partial(jax.jit, static_argnums=(2, 3))
def _gather_rows(idx3, table, per_w, d):
    total = _NW * per_w * _CHUNK
    mesh = plsc.VectorSubcoreMesh(core_axis_name="c", subcore_axis_name="s")

    @functools.partial(
        pl.kernel,
        out_type=jax.ShapeDtypeStruct((total, d), jnp.float32),
        mesh=mesh,
        scratch_types=[
            pltpu.VMEM((per_w, _CHUNK), jnp.int32),
            pltpu.VMEM((_CHUNK, d), jnp.float32),
            pltpu.SemaphoreType.DMA,
        ],
        compiler_params=pltpu.CompilerParams(use_tc_tiling_on_sc=False),
    )
    def emb(table_hbm, idx_hbm, out_hbm, idx_v, rows_v, gsem):
        wid = lax.axis_index("s") * 2 + lax.axis_index("c")
        base = wid * (per_w * _CHUNK)
        pltpu.sync_copy(idx_hbm.at[wid], idx_v)

        def step(j, carry):
            pltpu.async_copy(table_hbm.at[idx_v.at[j]], rows_v, gsem).wait()
            pltpu.sync_copy(
                rows_v, out_hbm.at[pl.ds(base + j * _CHUNK, _CHUNK)]
            )
            return carry

        lax.fori_loop(0, per_w, step, None)

    return emb(table, idx3)


def kernel(indices, table):
    b, h = indices.shape
    _, d = table.shape
    total = b * h
    per_w = total // (_NW * _CHUNK)
    assert per_w * _NW * _CHUNK == total
    idx3 = indices.reshape(_NW, per_w, _CHUNK)
    rows = _gather_rows(idx3, table, per_w, d)
    return rows.reshape(b, h, d)


# speedup vs baseline: 4.6668x; 1.1424x over previous
import functools

import jax
import jax.numpy as jnp
from jax import lax
from jax.experimental import pallas as pl
from jax.experimental.pallas import tpu as pltpu
from jax.experimental.pallas import tpu_sc as plsc

_NW = 32
_CHUNK = 128
_NBUF = 5


@functools.partial(jax.jit, static_argnums=(2, 3))
def _gather_rows(idx3, table, per_w, d):
    total = _NW * per_w * _CHUNK
    n_groups = per_w // _NBUF
    assert n_groups * _NBUF == per_w
    mesh = plsc.VectorSubcoreMesh(core_axis_name="c", subcore_axis_name="s")

    @functools.partial(
        pl.kernel,
        out_type=jax.ShapeDtypeStruct((total, d), jnp.float32),
        mesh=mesh,
        scratch_types=[
            pltpu.VMEM((per_w, _CHUNK), jnp.int32),
            pltpu.VMEM((_NBUF, _CHUNK, d), jnp.float32),
            [pltpu.SemaphoreType.DMA] * _NBUF,
            [pltpu.SemaphoreType.DMA] * _NBUF,
        ],
        compiler_params=pltpu.CompilerParams(use_tc_tiling_on_sc=False),
    )
    def emb(table_hbm, idx_hbm, out_hbm, idx_v, rows_v, gsems, wsems):
        wid = lax.axis_index("s") * 2 + lax.axis_index("c")
        base = wid * (per_w * _CHUNK)
        pltpu.sync_copy(idx_hbm.at[wid], idx_v)

        def start_gather(j, b):
            pltpu.async_copy(table_hbm.at[idx_v.at[j]], rows_v.at[b], gsems[b])

        def wait_gather(j, b):
            pltpu.make_async_copy(
                table_hbm.at[idx_v.at[j]], rows_v.at[b], gsems[b]
            ).wait()

        def start_write(j, b):
            pltpu.async_copy(
                rows_v.at[b],
                out_hbm.at[pl.ds(base + j * _CHUNK, _CHUNK)],
                wsems[b],
            )

        def wait_write(j, b):
            pltpu.make_async_copy(
                rows_v.at[b],
                out_hbm.at[pl.ds(base + j * _CHUNK, _CHUNK)],
                wsems[b],
            ).wait()

        for b in range(_NBUF):
            start_gather(b, b)

        def group(g, carry):
            j0 = g * _NBUF
            for b in range(_NBUF):
                wait_gather(j0 + b, b)
                start_write(j0 + b, b)
            for b in range(_NBUF):
                wait_write(j0 + b, b)

                @pl.when(g < n_groups - 1)
                def _():
                    start_gather(j0 + b + _NBUF, b)

            return carry

        lax.fori_loop(0, n_groups, group, None)

    return emb(table, idx3)


def kernel(indices, table):
    b, h = indices.shape
    _, d = table.shape
    total = b * h
    per_w = total // (_NW * _CHUNK)
    assert per_w * _NW * _CHUNK == total
    idx3 = indices.reshape(_NW, per_w, _CHUNK)
    rows = _gather_rows(idx3, table, per_w, d)
    return rows.reshape(b, h, d)
